# baseline (speedup 1.0000x reference)
import functools

import jax
import jax.numpy as jnp
from jax import lax
from jax.experimental import pallas as pl
from jax.experimental.pallas import tpu as pltpu
from jax.experimental.pallas import tpu_sc as plsc

NC, NS, LANES = 2, 16, 16
N_NODES = 10000
N_EDGES = 320000
STATIC_B = N_NODES // 2
D = 128
HALF = N_NODES // NC

EPW = N_EDGES // NS
SCAN = 2048
TAIL = EPW - (EPW // SCAN) * SCAN
CHUNK = 128
MINI = TAIL - (TAIL // CHUNK) * CHUNK
ACC_ROWS = 5120
RPS = ACC_ROWS // NS
DUMP = STATIC_B
XRPS = HALF // NS

_mesh = plsc.VectorSubcoreMesh(
    core_axis_name="c", subcore_axis_name="s", num_cores=NC, num_subcores=NS
)


@functools.partial(
    pl.kernel,
    out_type=jax.ShapeDtypeStruct((NC, ACC_ROWS, D), jnp.float32),
    mesh=_mesh,
    scratch_types=[
        pltpu.VMEM((SCAN,), jnp.int32),
        pltpu.VMEM((SCAN,), jnp.int32),
        pltpu.VMEM((SCAN,), jnp.int32),
        pltpu.VMEM((SCAN,), jnp.int32),
        pltpu.VMEM((2, CHUNK, D), jnp.float32),
        pltpu.VMEM((8, D), jnp.float32),
        pltpu.VMEM((LANES,), jnp.int32),
        pltpu.VMEM_SHARED((HALF, D), jnp.float32),
        pltpu.VMEM_SHARED((ACC_ROWS, D), jnp.float32),
        pltpu.SemaphoreType.DMA,
        pltpu.SemaphoreType.DMA,
        pltpu.SemaphoreType.DMA,
        pltpu.SemaphoreType.DMA,
        pltpu.SemaphoreType.DMA,
        pltpu.SemaphoreType.DMA,
    ],
)
def _sc_agg(x_hbm, src_hbm, dst_hbm, bs_hbm, p_hbm,
            sbuf_src0, sbuf_src1, sbuf_dst0, sbuf_dst1, grow, zbuf, bsbuf,
            x_sp, acc, sema, semb, sem0, sem1, semc, semd):
    sbuf_src = (sbuf_src0, sbuf_src1)
    sbuf_dst = (sbuf_dst0, sbuf_dst1)
    c = lax.axis_index("c")
    s = lax.axis_index("s")
    x0 = c * HALF

    xcopy = pltpu.async_copy(
        x_hbm.at[pl.ds(x0 + s * XRPS, XRPS)],
        x_sp.at[pl.ds(s * XRPS, XRPS)], sem0)
    xtail = pltpu.async_copy(
        x_hbm.at[pl.ds(x0 + NS * XRPS, HALF - NS * XRPS)],
        x_sp.at[pl.ds(NS * XRPS, HALF - NS * XRPS)], sem1)

    zeros16 = jnp.zeros((LANES,), jnp.float32)
    for r in range(8):
        for q in range(D // LANES):
            zbuf[r, pl.ds(q * LANES, LANES)] = zeros16
    r0 = s * RPS

    def _zcopy(k, carry):
        pltpu.sync_copy(zbuf, acc.at[pl.ds(r0 + k * 8, 8)])
        return carry

    lax.fori_loop(0, RPS // 8, _zcopy, 0)

    pltpu.sync_copy(bs_hbm, bsbuf)
    bsv = jnp.minimum(bsbuf[...], jnp.int32(STATIC_B))

    xcopy.wait()
    xtail.wait()
    plsc.subcore_barrier()

    ebase = s * EPW
    stage_sizes = [SCAN] * (EPW // SCAN) + ([TAIL] if TAIL else [])

    def _stage_refs(t, size):
        slot = t % 2
        off = ebase + t * SCAN
        return ((src_hbm.at[pl.ds(off, size)],
                 sbuf_src[slot].at[pl.ds(0, size)], sema),
                (dst_hbm.at[pl.ds(off, size)],
                 sbuf_dst[slot].at[pl.ds(0, size)], semb))

    def _stage_in(t, size):
        for refs in _stage_refs(t, size):
            pltpu.async_copy(*refs)

    def _stage_wait(t, size):
        for refs in _stage_refs(t, size):
            pltpu.make_async_copy(*refs).wait()

    def _gidx(slot, j, n):
        return sbuf_src[slot].at[pl.ds(j * CHUNK, n)]

    def _sidx(slot, j, n):
        return sbuf_dst[slot].at[pl.ds(j * CHUNK, n)]

    def _gather(slot, j, gslot, sem, n=CHUNK):
        return pltpu.async_copy(
            x_sp.at[_gidx(slot, j, n)], grow.at[gslot, pl.ds(0, n)], sem)

    def _gwait(slot, j, gslot, sem, n=CHUNK):
        pltpu.make_async_copy(
            x_sp.at[_gidx(slot, j, n)], grow.at[gslot, pl.ds(0, n)], sem
        ).wait()

    def _scat(slot, j, gslot, n=CHUNK):
        pltpu.sync_copy(grow.at[gslot, pl.ds(0, n)],
                        acc.at[_sidx(slot, j, n)], add=True)

    _stage_in(0, stage_sizes[0])
    gsem = (sem0, sem1)

    pos = lax.iota(jnp.int32, LANES)

    def _mask_chunk(slot, j, nv=CHUNK // LANES):
        for q in range(nv):
            i0 = j * CHUNK + q * LANES
            vs = sbuf_src[slot][pl.ds(i0, LANES)]
            vd = sbuf_dst[slot][pl.ds(i0, LANES)]
            ls = vs - x0
            in_half = (ls >= 0) & (ls < HALF)
            m = (vd < bsv) & in_half
            gspread = jnp.where(in_half, ls, ls & 4095)
            dspread = DUMP + ((pos + i0 + s) & 63)
            sbuf_src[slot][pl.ds(i0, LANES)] = jnp.where(m, ls, gspread)
            sbuf_dst[slot][pl.ds(i0, LANES)] = jnp.where(m, vd, dspread)

    def _ascat(slot, j, gslot, sem, n=CHUNK):
        pltpu.async_copy(grow.at[gslot, pl.ds(0, n)],
                         acc.at[_sidx(slot, j, n)], sem, add=True)

    def _swait(slot, j, gslot, sem, n=CHUNK):
        pltpu.make_async_copy(grow.at[gslot, pl.ds(0, n)],
                              acc.at[_sidx(slot, j, n)], sem).wait()

    for t, size in enumerate(stage_sizes):
        slot = t % 2
        _stage_wait(t, size)
        if t + 1 < len(stage_sizes):
            _stage_in(t + 1, stage_sizes[t + 1])

        nch = size // CHUNK
        _mask_chunk(slot, 0)
        _mask_chunk(slot, 1)
        _gather(slot, 0, 0, sem0)
        _gather(slot, 1, 1, sem1)

        def _pipe(k, carry):
            j = 2 * k
            _mask_chunk(slot, j + 2)
            _mask_chunk(slot, j + 3)
            _gwait(slot, j, 0, sem0)
            _ascat(slot, j, 0, semc)
            _gwait(slot, j + 1, 1, sem1)
            _ascat(slot, j + 1, 1, semd)
            _swait(slot, j, 0, semc)
            _gather(slot, j + 2, 0, sem0)
            _swait(slot, j + 1, 1, semd)
            _gather(slot, j + 3, 1, sem1)
            return carry

        lax.fori_loop(0, nch // 2 - 1, _pipe, 0)

        _gwait(slot, nch - 2, 0, sem0)
        _scat(slot, nch - 2, 0)
        _gwait(slot, nch - 1, 1, sem1)
        _scat(slot, nch - 1, 1)

        rem = size - nch * CHUNK
        if rem:
            _mask_chunk(slot, nch, nv=rem // LANES)
            _gather(slot, nch, 0, sem0, n=rem)
            _gwait(slot, nch, 0, sem0, n=rem)
            _scat(slot, nch, 0, n=rem)

    plsc.subcore_barrier()

    pltpu.sync_copy(acc.at[pl.ds(r0, RPS)], p_hbm.at[c, pl.ds(r0, RPS)])


def _combine(x, p):
    blk = 1000

    def body(x_ref, p_ref, o_ref):
        o_ref[:, :D] = x_ref[...]
        o_ref[:, D:] = p_ref[0] + p_ref[1]

    return pl.pallas_call(
        body,
        grid=(STATIC_B // blk,),
        in_specs=[
            pl.BlockSpec((blk, D), lambda i: (i, 0)),
            pl.BlockSpec((NC, blk, D), lambda i: (0, i, 0)),
        ],
        out_specs=pl.BlockSpec((blk, 2 * D), lambda i: (i, 0)),
        out_shape=jax.ShapeDtypeStruct((STATIC_B, 2 * D), jnp.float32),
    )(x, p)


def kernel(x, edge_index, batch_size):
    ei = edge_index.astype(jnp.int32)
    bsv = jnp.full((LANES,), batch_size, jnp.int32)
    p = _sc_agg(x, ei[0], ei[1], bsv)
    return _combine(x, p)


# speedup vs baseline: 1.2132x; 1.2132x over previous
import functools

import jax
import jax.numpy as jnp
from jax import lax
from jax.experimental import pallas as pl
from jax.experimental.pallas import tpu as pltpu
from jax.experimental.pallas import tpu_sc as plsc

NC, NS, LANES = 2, 16, 16
N_NODES = 10000
N_EDGES = 320000
STATIC_B = N_NODES // 2
D = 128
HALF = N_NODES // NC

EPW = N_EDGES // NS
SCAN = 2048
TAIL = EPW - (EPW // SCAN) * SCAN
CHUNK = 128
MINI = TAIL - (TAIL // CHUNK) * CHUNK
ACC_ROWS = 5120
RPS = ACC_ROWS // NS
DUMP = STATIC_B
XRPS = HALF // NS

_mesh = plsc.VectorSubcoreMesh(
    core_axis_name="c", subcore_axis_name="s", num_cores=NC, num_subcores=NS
)


@functools.partial(
    pl.kernel,
    out_type=jax.ShapeDtypeStruct((NC, ACC_ROWS, D), jnp.float32),
    mesh=_mesh,
    scratch_types=[
        pltpu.VMEM((SCAN,), jnp.int32),
        pltpu.VMEM((SCAN,), jnp.int32),
        pltpu.VMEM((SCAN,), jnp.int32),
        pltpu.VMEM((SCAN,), jnp.int32),
        pltpu.VMEM((2, CHUNK, D), jnp.float32),
        pltpu.VMEM((8, D), jnp.float32),
        pltpu.VMEM((LANES,), jnp.int32),
        pltpu.VMEM_SHARED((HALF, D), jnp.float32),
        pltpu.VMEM_SHARED((ACC_ROWS, D), jnp.float32),
        pltpu.SemaphoreType.DMA,
        pltpu.SemaphoreType.DMA,
        pltpu.SemaphoreType.DMA,
        pltpu.SemaphoreType.DMA,
        pltpu.SemaphoreType.DMA,
        pltpu.SemaphoreType.DMA,
    ],
)
def _sc_agg(x_hbm, src_hbm, dst_hbm, bs_hbm, p_hbm,
            sbuf_src0, sbuf_src1, sbuf_dst0, sbuf_dst1, grow, zbuf, bsbuf,
            x_sp, acc, sema, semb, sem0, sem1, semc, semd):
    sbuf_src = (sbuf_src0, sbuf_src1)
    sbuf_dst = (sbuf_dst0, sbuf_dst1)
    c = lax.axis_index("c")
    s = lax.axis_index("s")
    x0 = c * HALF

    xcopy = pltpu.async_copy(
        x_hbm.at[pl.ds(x0 + s * XRPS, XRPS)],
        x_sp.at[pl.ds(s * XRPS, XRPS)], sem0)
    xtail = pltpu.async_copy(
        x_hbm.at[pl.ds(x0 + NS * XRPS, HALF - NS * XRPS)],
        x_sp.at[pl.ds(NS * XRPS, HALF - NS * XRPS)], sem1)

    zeros16 = jnp.zeros((LANES,), jnp.float32)
    for r in range(8):
        for q in range(D // LANES):
            zbuf[r, pl.ds(q * LANES, LANES)] = zeros16
    r0 = s * RPS

    def _zcopy(k, carry):
        pltpu.sync_copy(zbuf, acc.at[pl.ds(r0 + k * 8, 8)])
        return carry

    lax.fori_loop(0, RPS // 8, _zcopy, 0)

    pltpu.sync_copy(bs_hbm, bsbuf)
    bsv = jnp.minimum(bsbuf[...], jnp.int32(STATIC_B))

    xcopy.wait()
    xtail.wait()
    plsc.subcore_barrier()

    ebase = s * EPW
    stage_sizes = [SCAN] * (EPW // SCAN) + ([TAIL] if TAIL else [])

    def _stage_refs(t, size):
        slot = t % 2
        off = ebase + t * SCAN
        return ((src_hbm.at[pl.ds(off, size)],
                 sbuf_src[slot].at[pl.ds(0, size)], sema),
                (dst_hbm.at[pl.ds(off, size)],
                 sbuf_dst[slot].at[pl.ds(0, size)], semb))

    def _stage_in(t, size):
        for refs in _stage_refs(t, size):
            pltpu.async_copy(*refs)

    def _stage_wait(t, size):
        for refs in _stage_refs(t, size):
            pltpu.make_async_copy(*refs).wait()

    def _gidx(slot, j, n):
        return sbuf_src[slot].at[pl.ds(j * CHUNK, n)]

    def _sidx(slot, j, n):
        return sbuf_dst[slot].at[pl.ds(j * CHUNK, n)]

    def _gather(slot, j, gslot, sem, n=CHUNK):
        return pltpu.async_copy(
            x_sp.at[_gidx(slot, j, n)], grow.at[gslot, pl.ds(0, n)], sem)

    def _gwait(slot, j, gslot, sem, n=CHUNK):
        pltpu.make_async_copy(
            x_sp.at[_gidx(slot, j, n)], grow.at[gslot, pl.ds(0, n)], sem
        ).wait()

    def _scat(slot, j, gslot, n=CHUNK):
        pltpu.sync_copy(grow.at[gslot, pl.ds(0, n)],
                        acc.at[_sidx(slot, j, n)], add=True)

    _stage_in(0, stage_sizes[0])
    gsem = (sem0, sem1)

    pos = lax.iota(jnp.int32, LANES)

    def _mask_chunk(slot, j, nv=CHUNK // LANES):
        for q in range(nv):
            i0 = j * CHUNK + q * LANES
            vs = sbuf_src[slot][pl.ds(i0, LANES)]
            vd = sbuf_dst[slot][pl.ds(i0, LANES)]
            ls = vs - x0
            in_half = (ls >= 0) & (ls < HALF)
            m = (vd < bsv) & in_half
            gspread = jnp.where(in_half, ls, ls & 4095)
            dspread = DUMP + ((pos + i0 + s) & 63)
            sbuf_src[slot][pl.ds(i0, LANES)] = jnp.where(m, ls, gspread)
            sbuf_dst[slot][pl.ds(i0, LANES)] = jnp.where(m, vd, dspread)

    def _ascat(slot, j, gslot, sem, n=CHUNK):
        pltpu.async_copy(grow.at[gslot, pl.ds(0, n)],
                         acc.at[_sidx(slot, j, n)], sem, add=True)

    def _swait(slot, j, gslot, sem, n=CHUNK):
        pltpu.make_async_copy(grow.at[gslot, pl.ds(0, n)],
                              acc.at[_sidx(slot, j, n)], sem).wait()

    for t, size in enumerate(stage_sizes):
        slot = t % 2
        _stage_wait(t, size)
        if t + 1 < len(stage_sizes):
            _stage_in(t + 1, stage_sizes[t + 1])

        nch = size // CHUNK
        _mask_chunk(slot, 0)
        _mask_chunk(slot, 1)
        _gather(slot, 0, 0, sem0)
        _gather(slot, 1, 1, sem1)

        def _pipe(k, carry):
            j = 2 * k
            _mask_chunk(slot, j + 2)
            _gwait(slot, j, 0, sem0)
            _scat(slot, j, 0)
            _gather(slot, j + 2, 0, sem0)
            _mask_chunk(slot, j + 3)
            _gwait(slot, j + 1, 1, sem1)
            _scat(slot, j + 1, 1)
            _gather(slot, j + 3, 1, sem1)
            return carry

        lax.fori_loop(0, nch // 2 - 1, _pipe, 0)

        _gwait(slot, nch - 2, 0, sem0)
        _scat(slot, nch - 2, 0)
        _gwait(slot, nch - 1, 1, sem1)
        _scat(slot, nch - 1, 1)

        rem = size - nch * CHUNK
        if rem:
            _mask_chunk(slot, nch, nv=rem // LANES)
            _gather(slot, nch, 0, sem0, n=rem)
            _gwait(slot, nch, 0, sem0, n=rem)
            _scat(slot, nch, 0, n=rem)

    plsc.subcore_barrier()

    pltpu.sync_copy(acc.at[pl.ds(r0, RPS)], p_hbm.at[c, pl.ds(r0, RPS)])


def _combine(x, p):
    blk = 1000

    def body(x_ref, p_ref, o_ref):
        o_ref[:, :D] = x_ref[...]
        o_ref[:, D:] = p_ref[0] + p_ref[1]

    return pl.pallas_call(
        body,
        grid=(STATIC_B // blk,),
        in_specs=[
            pl.BlockSpec((blk, D), lambda i: (i, 0)),
            pl.BlockSpec((NC, blk, D), lambda i: (0, i, 0)),
        ],
        out_specs=pl.BlockSpec((blk, 2 * D), lambda i: (i, 0)),
        out_shape=jax.ShapeDtypeStruct((STATIC_B, 2 * D), jnp.float32),
    )(x, p)


def kernel(x, edge_index, batch_size):
    ei = edge_index.astype(jnp.int32)
    bsv = jnp.full((LANES,), batch_size, jnp.int32)
    p = _sc_agg(x, ei[0], ei[1], bsv)
    return _combine(x, p)
